# baseline (speedup 1.0000x reference)
import functools

import jax
import jax.numpy as jnp
from jax import lax
from jax.experimental import pallas as pl
from jax.experimental.pallas import tpu as pltpu
from jax.experimental.pallas import tpu_sc as plsc

BATCH = 16384
N_HID = 64
N_TAB = 4
LANES = 16
NUM_CORES = 2
NUM_SUBCORES = 16
NW = NUM_CORES * NUM_SUBCORES
BPW = BATCH // NW
GROUP = 4
GBUF = N_TAB * GROUP
NGRP = BPW // GROUP
IDXLEN = BPW + LANES

_mesh = plsc.VectorSubcoreMesh(core_axis_name="c", subcore_axis_name="s")


@functools.partial(
    pl.kernel,
    mesh=_mesh,
    out_type=jax.ShapeDtypeStruct((BATCH, N_HID), jnp.float32),
    scratch_types=[
        pltpu.VMEM((N_TAB, IDXLEN), jnp.int32),
        pltpu.VMEM((GBUF, N_HID), jnp.float32),
        pltpu.VMEM((GBUF, N_HID), jnp.float32),
        pltpu.VMEM((BPW, N_HID), jnp.float32),
        pltpu.SemaphoreType.DMA,
        pltpu.SemaphoreType.DMA,
    ],
)
def _lookup_sum(tT, e0, e1, e2, e3, out, idx_v, rb0, rb1, obuf, sm0, sm1):
    wid = lax.axis_index("s") * NUM_CORES + lax.axis_index("c")
    base = wid * BPW
    tabs = (e0, e1, e2, e3)
    rbs = (rb0, rb1)
    sms = (sm0, sm1)

    for k in range(N_TAB):
        pltpu.sync_copy(tT.at[k, pl.ds(base, BPW)], idx_v.at[k, pl.ds(0, BPW)])
    zeros = jnp.zeros((LANES,), jnp.int32)
    for k in range(N_TAB):
        idx_v[k, pl.ds(BPW, LANES)] = zeros

    def enqueue(base16, lo, rbuf, sem):
        iv = [idx_v[k, pl.ds(base16, LANES)] for k in range(N_TAB)]
        for k in range(N_TAB):
            for r in range(GROUP):
                pltpu.async_copy(tabs[k].at[iv[k][lo + r]],
                                 rbuf.at[k * GROUP + r], sem)

    def drain(rbuf, sem):
        pltpu.make_async_copy(e0.at[pl.ds(0, GBUF)], rbuf, sem).wait()

    def vsum(g, rbuf):
        row0 = g * GROUP
        for r in range(GROUP):
            for j in range(N_HID // LANES):
                o = j * LANES
                v = (rbuf[0 * GROUP + r, pl.ds(o, LANES)]
                     + rbuf[1 * GROUP + r, pl.ds(o, LANES)]
                     + rbuf[2 * GROUP + r, pl.ds(o, LANES)]
                     + rbuf[3 * GROUP + r, pl.ds(o, LANES)])
                obuf[row0 + r, pl.ds(o, LANES)] = v

    PER = LANES // GROUP
    enqueue(0, 0, rb0, sm0)

    def vec_body(gg, _):
        b = gg * PER
        for u in range(PER):
            g = b + u
            enqueue((gg + (u + 1) // PER) * LANES,
                    GROUP * ((u + 1) % PER),
                    rbs[(u + 1) % 2], sms[(u + 1) % 2])
            drain(rbs[u % 2], sms[u % 2])
            vsum(g, rbs[u % 2])
        return 0

    lax.fori_loop(0, NGRP // PER, vec_body, 0)
    drain(rbs[NGRP % 2], sms[NGRP % 2])
    pltpu.sync_copy(obuf, out.at[pl.ds(base, BPW)])


def kernel(t, emb0, emb1, emb2, emb3):
    tT = t.T.reshape(N_TAB, BATCH)
    return _lookup_sum(tT, emb0, emb1, emb2, emb3)
